# Initial kernel scaffold; baseline (speedup 1.0000x reference)
import jax
import jax.numpy as jnp
from jax.experimental import pallas as pl


def kernel(embeddings, segment_types, positions, seg_table, pos_table, ln_gamma, ln_beta):
    raise NotImplementedError("write your pallas kernel here")


# speedup vs baseline: 2.0979x; 2.0979x over previous
import functools

import jax
import jax.numpy as jnp
from jax import lax
from jax.experimental import pallas as pl
from jax.experimental.pallas import tpu as pltpu

_D = 256
_SEG_PAD = 8
_POS_PAD = 32


def _fused_kernel(emb_ref, seg_ref, pos_ref, st_ref, pt_ref, g_ref, b_ref,
                  out_ref):
    emb = emb_ref[...]
    seg = seg_ref[0]
    pos = pos_ref[0]

    r = emb.shape[0]
    oh_s = (seg == lax.broadcasted_iota(jnp.int32, (r, _SEG_PAD), 1))
    oh_p = (pos == lax.broadcasted_iota(jnp.int32, (r, _POS_PAD), 1))
    oh_s = oh_s.astype(jnp.bfloat16)
    oh_p = oh_p.astype(jnp.bfloat16)

    st = st_ref[...]
    pt = pt_ref[...]
    st_hi = st.astype(jnp.bfloat16)
    st_lo = (st - st_hi.astype(jnp.float32)).astype(jnp.bfloat16)
    pt_hi = pt.astype(jnp.bfloat16)
    pt_lo = (pt - pt_hi.astype(jnp.float32)).astype(jnp.bfloat16)

    x = emb
    x = x + jnp.dot(oh_s, st_hi, preferred_element_type=jnp.float32)
    x = x + jnp.dot(oh_s, st_lo, preferred_element_type=jnp.float32)
    x = x + jnp.dot(oh_p, pt_hi, preferred_element_type=jnp.float32)
    x = x + jnp.dot(oh_p, pt_lo, preferred_element_type=jnp.float32)

    mean = jnp.sum(x, axis=1, keepdims=True) * (1.0 / _D)
    meansq = jnp.sum(x * x, axis=1, keepdims=True) * (1.0 / _D)
    var = meansq - mean * mean
    inv = lax.rsqrt(var + 1e-5)
    gamma = g_ref[...]
    beta = b_ref[...]
    scale = inv * gamma
    shift = beta - (mean * inv) * gamma
    out_ref[...] = x * scale + shift


@functools.partial(jax.jit, static_argnames=("block_rows",))
def _run(emb2d, seg3d, pos3d, st_pad, pt_pad, gamma2d, beta2d,
         block_rows=512):
    n = emb2d.shape[0]
    nb = n // block_rows
    grid = (nb,)
    out = pl.pallas_call(
        _fused_kernel,
        grid=grid,
        in_specs=[
            pl.BlockSpec((block_rows, _D), lambda i: (i, 0)),
            pl.BlockSpec((1, block_rows, 1), lambda i: (i, 0, 0)),
            pl.BlockSpec((1, block_rows, 1), lambda i: (i, 0, 0)),
            pl.BlockSpec((_SEG_PAD, _D), lambda i: (0, 0)),
            pl.BlockSpec((_POS_PAD, _D), lambda i: (0, 0)),
            pl.BlockSpec((1, _D), lambda i: (0, 0)),
            pl.BlockSpec((1, _D), lambda i: (0, 0)),
        ],
        out_specs=pl.BlockSpec((block_rows, _D), lambda i: (i, 0)),
        out_shape=jax.ShapeDtypeStruct((n, _D), jnp.float32),
        compiler_params=pltpu.CompilerParams(
            dimension_semantics=("arbitrary",),
        ),
    )(emb2d, seg3d, pos3d, st_pad, pt_pad, gamma2d, beta2d)
    return out


def kernel(embeddings, segment_types, positions, seg_table, pos_table,
           ln_gamma, ln_beta):
    b, t, d = embeddings.shape
    n = b * t
    block_rows = 512
    nb = n // block_rows

    emb2d = embeddings.reshape(n, d)
    seg3d = segment_types.astype(jnp.int32).reshape(nb, block_rows, 1)
    pos3d = positions.astype(jnp.int32).reshape(nb, block_rows, 1)
    st_pad = jnp.zeros((_SEG_PAD, d), jnp.float32).at[:seg_table.shape[0]].set(
        seg_table)
    pt_pad = jnp.zeros((_POS_PAD, d), jnp.float32).at[:pos_table.shape[0]].set(
        pos_table)
    gamma2d = ln_gamma.reshape(1, d)
    beta2d = ln_beta.reshape(1, d)

    out = _run(emb2d, seg3d, pos3d, st_pad, pt_pad, gamma2d, beta2d,
               block_rows=block_rows)
    return out.reshape(b, t, d)


# speedup vs baseline: 2.4734x; 1.1790x over previous
import functools

import jax
import jax.numpy as jnp
from jax import lax
from jax.experimental import pallas as pl
from jax.experimental.pallas import tpu as pltpu

_D = 256
_SEG_PAD = 8
_POS_PAD = 32


def _fused_kernel(emb_ref, seg_ref, pos_ref, st_ref, pt_ref, g_ref, b_ref,
                  out_ref):
    emb = emb_ref[...]
    seg = seg_ref[0]
    pos = pos_ref[0]

    r = emb.shape[0]
    oh_s = (seg == lax.broadcasted_iota(jnp.int32, (r, _SEG_PAD), 1))
    oh_p = (pos == lax.broadcasted_iota(jnp.int32, (r, _POS_PAD), 1))
    oh_s = oh_s.astype(jnp.bfloat16)
    oh_p = oh_p.astype(jnp.bfloat16)

    st = st_ref[...]
    pt = pt_ref[...]
    st_hi = st.astype(jnp.bfloat16)
    st_lo = (st - st_hi.astype(jnp.float32)).astype(jnp.bfloat16)
    pt_hi = pt.astype(jnp.bfloat16)
    pt_lo = (pt - pt_hi.astype(jnp.float32)).astype(jnp.bfloat16)

    x = emb
    x = x + jnp.dot(oh_s, st_hi, preferred_element_type=jnp.float32)
    x = x + jnp.dot(oh_s, st_lo, preferred_element_type=jnp.float32)
    x = x + jnp.dot(oh_p, pt_hi, preferred_element_type=jnp.float32)
    x = x + jnp.dot(oh_p, pt_lo, preferred_element_type=jnp.float32)

    mean = jnp.sum(x, axis=1, keepdims=True) * (1.0 / _D)
    meansq = jnp.sum(x * x, axis=1, keepdims=True) * (1.0 / _D)
    var = meansq - mean * mean
    inv = lax.rsqrt(var + 1e-5)
    gamma = g_ref[...]
    beta = b_ref[...]
    scale = inv * gamma
    shift = beta - (mean * inv) * gamma
    out_ref[...] = x * scale + shift


@functools.partial(jax.jit, static_argnames=("block_rows",))
def _run(emb2d, seg3d, pos3d, st_pad, pt_pad, gamma2d, beta2d,
         block_rows=512):
    n = emb2d.shape[0]
    nb = n // block_rows
    grid = (nb,)
    out = pl.pallas_call(
        _fused_kernel,
        grid=grid,
        in_specs=[
            pl.BlockSpec((block_rows, _D), lambda i: (i, 0)),
            pl.BlockSpec((1, block_rows, 1), lambda i: (i, 0, 0)),
            pl.BlockSpec((1, block_rows, 1), lambda i: (i, 0, 0)),
            pl.BlockSpec((_SEG_PAD, _D), lambda i: (0, 0)),
            pl.BlockSpec((_POS_PAD, _D), lambda i: (0, 0)),
            pl.BlockSpec((1, _D), lambda i: (0, 0)),
            pl.BlockSpec((1, _D), lambda i: (0, 0)),
        ],
        out_specs=pl.BlockSpec((block_rows, _D), lambda i: (i, 0)),
        out_shape=jax.ShapeDtypeStruct((n, _D), jnp.float32),
        compiler_params=pltpu.CompilerParams(
            dimension_semantics=("arbitrary",),
        ),
    )(emb2d, seg3d, pos3d, st_pad, pt_pad, gamma2d, beta2d)
    return out


def kernel(embeddings, segment_types, positions, seg_table, pos_table,
           ln_gamma, ln_beta):
    b, t, d = embeddings.shape
    n = b * t
    block_rows = 2048
    nb = n // block_rows

    emb2d = embeddings.reshape(n, d)
    seg3d = segment_types.astype(jnp.int32).reshape(nb, block_rows, 1)
    pos3d = positions.astype(jnp.int32).reshape(nb, block_rows, 1)
    st_pad = jnp.concatenate(
        [seg_table, jnp.zeros((_SEG_PAD - seg_table.shape[0], d), jnp.float32)])
    pt_pad = jnp.concatenate(
        [pos_table, jnp.zeros((_POS_PAD - pos_table.shape[0], d), jnp.float32)])
    gamma2d = ln_gamma.reshape(1, d)
    beta2d = ln_beta.reshape(1, d)

    out = _run(emb2d, seg3d, pos3d, st_pad, pt_pad, gamma2d, beta2d,
               block_rows=block_rows)
    return out.reshape(b, t, d)


# speedup vs baseline: 2.5615x; 1.0356x over previous
import functools

import jax
import jax.numpy as jnp
from jax import lax
from jax.experimental import pallas as pl
from jax.experimental.pallas import tpu as pltpu

_D = 256
_SEG_PAD = 8
_POS_PAD = 32


def _fused_kernel(emb_ref, seg_ref, pos_ref, st_ref, pt_ref, g_ref, b_ref,
                  out_ref):
    emb = emb_ref[...]
    seg = seg_ref[0]
    pos = pos_ref[0]

    r = emb.shape[0]
    oh_s = (seg == lax.broadcasted_iota(jnp.int32, (r, _SEG_PAD), 1))
    oh_p = (pos == lax.broadcasted_iota(jnp.int32, (r, _POS_PAD), 1))
    oh_s = oh_s.astype(jnp.bfloat16)
    oh_p = oh_p.astype(jnp.bfloat16)

    st = st_ref[...]
    pt = pt_ref[...]
    st_hi = st.astype(jnp.bfloat16)
    st_lo = (st - st_hi.astype(jnp.float32)).astype(jnp.bfloat16)
    pt_hi = pt.astype(jnp.bfloat16)
    pt_lo = (pt - pt_hi.astype(jnp.float32)).astype(jnp.bfloat16)

    x = emb
    x = x + jnp.dot(oh_s, st_hi, preferred_element_type=jnp.float32)
    x = x + jnp.dot(oh_s, st_lo, preferred_element_type=jnp.float32)
    x = x + jnp.dot(oh_p, pt_hi, preferred_element_type=jnp.float32)
    x = x + jnp.dot(oh_p, pt_lo, preferred_element_type=jnp.float32)

    mean = jnp.sum(x, axis=1, keepdims=True) * (1.0 / _D)
    meansq = jnp.sum(x * x, axis=1, keepdims=True) * (1.0 / _D)
    var = meansq - mean * mean
    inv = lax.rsqrt(var + 1e-5)
    gamma = g_ref[...]
    beta = b_ref[...]
    scale = inv * gamma
    shift = beta - (mean * inv) * gamma
    out_ref[...] = x * scale + shift


@functools.partial(jax.jit, static_argnames=("block_rows",))
def _run(emb2d, seg3d, pos3d, st_pad, pt_pad, gamma2d, beta2d,
         block_rows=512):
    n = emb2d.shape[0]
    nb = n // block_rows
    grid = (nb,)
    out = pl.pallas_call(
        _fused_kernel,
        grid=grid,
        in_specs=[
            pl.BlockSpec((block_rows, _D), lambda i: (i, 0)),
            pl.BlockSpec((1, block_rows, 1), lambda i: (i, 0, 0)),
            pl.BlockSpec((1, block_rows, 1), lambda i: (i, 0, 0)),
            pl.BlockSpec((_SEG_PAD, _D), lambda i: (0, 0)),
            pl.BlockSpec((_POS_PAD, _D), lambda i: (0, 0)),
            pl.BlockSpec((1, _D), lambda i: (0, 0)),
            pl.BlockSpec((1, _D), lambda i: (0, 0)),
        ],
        out_specs=pl.BlockSpec((block_rows, _D), lambda i: (i, 0)),
        out_shape=jax.ShapeDtypeStruct((n, _D), jnp.float32),
        compiler_params=pltpu.CompilerParams(
            dimension_semantics=("arbitrary",),
        ),
    )(emb2d, seg3d, pos3d, st_pad, pt_pad, gamma2d, beta2d)
    return out


def kernel(embeddings, segment_types, positions, seg_table, pos_table,
           ln_gamma, ln_beta):
    b, t, d = embeddings.shape
    n = b * t
    block_rows = 4096
    nb = n // block_rows

    emb2d = embeddings.reshape(n, d)
    seg3d = segment_types.astype(jnp.int32).reshape(nb, block_rows, 1)
    pos3d = positions.astype(jnp.int32).reshape(nb, block_rows, 1)
    st_pad = jnp.concatenate(
        [seg_table, jnp.zeros((_SEG_PAD - seg_table.shape[0], d), jnp.float32)])
    pt_pad = jnp.concatenate(
        [pos_table, jnp.zeros((_POS_PAD - pos_table.shape[0], d), jnp.float32)])
    gamma2d = ln_gamma.reshape(1, d)
    beta2d = ln_beta.reshape(1, d)

    out = _run(emb2d, seg3d, pos3d, st_pad, pt_pad, gamma2d, beta2d,
               block_rows=block_rows)
    return out.reshape(b, t, d)
